# baseline (speedup 1.0000x reference)
import functools

import jax
import jax.numpy as jnp
from jax import lax
from jax.experimental import pallas as pl
from jax.experimental.pallas import tpu as pltpu
from jax.experimental.pallas import tpu_sc as plsc

BATCH = 16384
D = 128
GATHER_WINDOW = 128
NUM_CHUNKS = 4
CHUNK = BATCH // NUM_CHUNKS
MLP_BLOCK = 2048


def _gather_body(n, uf_hbm, if_hbm, ui_hbm, ii_hbm, ue_hbm, ie_hbm):
    def body(ui_vmem, ii_vmem, ue_vmem, ie_vmem):
        def scoped(s1, s2):
            c1 = pltpu.make_async_copy(uf_hbm.at[ui_vmem.at[0]], ue_vmem, s1)
            c2 = pltpu.make_async_copy(if_hbm.at[ii_vmem.at[0]], ie_vmem, s2)
            c1.start()
            c2.start()
            c1.wait()
            c2.wait()

        pl.run_scoped(scoped, pltpu.SemaphoreType.DMA, pltpu.SemaphoreType.DMA)

    pltpu.emit_pipeline(
        body,
        grid=(n // GATHER_WINDOW,),
        in_specs=[
            pl.BlockSpec((1, GATHER_WINDOW), lambda i: (0, i)),
            pl.BlockSpec((1, GATHER_WINDOW), lambda i: (0, i)),
        ],
        out_specs=[
            pl.BlockSpec((GATHER_WINDOW, D), lambda i: (i, 0)),
            pl.BlockSpec((GATHER_WINDOW, D), lambda i: (i, 0)),
        ],
        core_axis_name=("c", "s"),
        dimension_semantics=(pltpu.PARALLEL,),
    )(ui_hbm, ii_hbm, ue_hbm, ie_hbm)


def _sc_gather(user_factors, item_factors, user2d, item2d):
    n = user2d.shape[1]
    mesh = plsc.VectorSubcoreMesh(core_axis_name="c", subcore_axis_name="s")
    f = pl.kernel(
        functools.partial(_gather_body, n),
        out_type=(
            jax.ShapeDtypeStruct((n, D), jnp.float32),
            jax.ShapeDtypeStruct((n, D), jnp.float32),
        ),
        mesh=mesh,
    )
    return f(user_factors, item_factors, user2d, item2d)


def _mlp_body(ue_ref, ie_ref, w1u_ref, w1i_ref, b1_ref, w2_ref, b2_ref,
              w3_ref, b3_ref, o_ref):
    dn = (((1,), (1,)), ((), ()))
    h = lax.dot_general(ue_ref[...], w1u_ref[...], dn,
                        preferred_element_type=jnp.float32)
    h = h + lax.dot_general(ie_ref[...], w1i_ref[...], dn,
                            preferred_element_type=jnp.float32)
    h = jnp.maximum(h + b1_ref[...][None, :], 0.0)
    h = lax.dot_general(h, w2_ref[...], dn, preferred_element_type=jnp.float32)
    h = jnp.maximum(h + b2_ref[...][None, :], 0.0)
    h = lax.dot_general(h, w3_ref[...], dn, preferred_element_type=jnp.float32)
    o_ref[...] = jax.nn.sigmoid(h + b3_ref[...][None, :])


def _tc_mlp(ue, ie, W1u, W1i, b1, W2, b2, W3, b3):
    n = ue.shape[0]
    blk = min(MLP_BLOCK, n)
    full = lambda shape: pl.BlockSpec(shape, lambda i: tuple(0 for _ in shape))
    return pl.pallas_call(
        _mlp_body,
        grid=(n // blk,),
        in_specs=[
            pl.BlockSpec((blk, D), lambda i: (i, 0)),
            pl.BlockSpec((blk, D), lambda i: (i, 0)),
            full(W1u.shape), full(W1i.shape), full(b1.shape),
            full(W2.shape), full(b2.shape), full(W3.shape), full(b3.shape),
        ],
        out_specs=pl.BlockSpec((blk, 2), lambda i: (i, 0)),
        out_shape=jax.ShapeDtypeStruct((n, 2), jnp.float32),
    )(ue, ie, W1u, W1i, b1, W2, b2, W3, b3)


def kernel(user, item, user_factors, item_factors, W1, b1, W2, b2, W3, b3):
    user2d = user.astype(jnp.int32).reshape(NUM_CHUNKS, 1, CHUNK)
    item2d = item.astype(jnp.int32).reshape(NUM_CHUNKS, 1, CHUNK)
    W1u = W1[:, :D]
    W1i = W1[:, D:]
    embs = [
        _sc_gather(user_factors, item_factors, user2d[c], item2d[c])
        for c in range(NUM_CHUNKS)
    ]
    outs = [
        _tc_mlp(ue, ie, W1u, W1i, b1, W2, b2, W3, b3) for ue, ie in embs
    ]
    return jnp.concatenate(outs, axis=0)


# speedup vs baseline: 1.0944x; 1.0944x over previous
import functools

import jax
import jax.numpy as jnp
from jax import lax
from jax.experimental import pallas as pl
from jax.experimental.pallas import tpu as pltpu
from jax.experimental.pallas import tpu_sc as plsc

BATCH = 16384
D = 128
GATHER_WINDOW = 128
NUM_CHUNKS = 2
WINDOWS_PER_STEP = 2
CHUNK = BATCH // NUM_CHUNKS
MLP_BLOCK = 2048


def _gather_body(chunk, uf_hbm, if_hbm, ui_hbm, ii_hbm, ue_hbm, ie_hbm):
    off = chunk * (CHUNK // GATHER_WINDOW)

    def body(ui_vmem, ii_vmem, ue_vmem, ie_vmem):
        def scoped(s1, s2):
            c1 = pltpu.make_async_copy(uf_hbm.at[ui_vmem.at[0]], ue_vmem, s1)
            c2 = pltpu.make_async_copy(if_hbm.at[ii_vmem.at[0]], ie_vmem, s2)
            c1.start()
            c2.start()
            c1.wait()
            c2.wait()

        pl.run_scoped(scoped, pltpu.SemaphoreType.DMA, pltpu.SemaphoreType.DMA)

    pltpu.emit_pipeline(
        body,
        grid=(CHUNK // GATHER_WINDOW,),
        in_specs=[
            pl.BlockSpec((1, GATHER_WINDOW), lambda i: (off + i, 0)),
            pl.BlockSpec((1, GATHER_WINDOW), lambda i: (off + i, 0)),
        ],
        out_specs=[
            pl.BlockSpec((GATHER_WINDOW, D), lambda i: (i, 0)),
            pl.BlockSpec((GATHER_WINDOW, D), lambda i: (i, 0)),
        ],
        core_axis_name=("c", "s"),
        dimension_semantics=(pltpu.PARALLEL,),
    )(ui_hbm, ii_hbm, ue_hbm, ie_hbm)


def _sc_gather(user_factors, item_factors, user2d, item2d, chunk):
    mesh = plsc.VectorSubcoreMesh(core_axis_name="c", subcore_axis_name="s")
    f = pl.kernel(
        functools.partial(_gather_body, chunk),
        out_type=(
            jax.ShapeDtypeStruct((CHUNK, D), jnp.float32),
            jax.ShapeDtypeStruct((CHUNK, D), jnp.float32),
        ),
        mesh=mesh,
    )
    return f(user_factors, item_factors, user2d, item2d)


def _mlp_body(ue_ref, ie_ref, w1u_ref, w1i_ref, b1_ref, w2_ref, b2_ref,
              w3_ref, b3_ref, o_ref):
    dn = (((1,), (1,)), ((), ()))
    h = lax.dot_general(ue_ref[...], w1u_ref[...], dn,
                        preferred_element_type=jnp.float32)
    h = h + lax.dot_general(ie_ref[...], w1i_ref[...], dn,
                            preferred_element_type=jnp.float32)
    h = jnp.maximum(h + b1_ref[...][None, :], 0.0)
    h = lax.dot_general(h, w2_ref[...], dn, preferred_element_type=jnp.float32)
    h = jnp.maximum(h + b2_ref[...][None, :], 0.0)
    h = lax.dot_general(h, w3_ref[...], dn, preferred_element_type=jnp.float32)
    o_ref[...] = jax.nn.sigmoid(h + b3_ref[...][None, :])


def _tc_mlp(ue, ie, W1, b1, W2, b2, W3, b3):
    n = ue.shape[0]
    blk = min(MLP_BLOCK, n)
    full = lambda shape: pl.BlockSpec(shape, lambda i: tuple(0 for _ in shape))
    return pl.pallas_call(
        _mlp_body,
        grid=(n // blk,),
        in_specs=[
            pl.BlockSpec((blk, D), lambda i: (i, 0)),
            pl.BlockSpec((blk, D), lambda i: (i, 0)),
            pl.BlockSpec((D, D), lambda i: (0, 0)),
        pl.BlockSpec((D, D), lambda i: (0, 1)),
        full(b1.shape),
            full(W2.shape), full(b2.shape), full(W3.shape), full(b3.shape),
        ],
        out_specs=pl.BlockSpec((blk, 2), lambda i: (i, 0)),
        out_shape=jax.ShapeDtypeStruct((n, 2), jnp.float32),
    )(ue, ie, W1, W1, b1, W2, b2, W3, b3)


def kernel(user, item, user_factors, item_factors, W1, b1, W2, b2, W3, b3):
    user2d = user.astype(jnp.int32).reshape(
        BATCH // GATHER_WINDOW, GATHER_WINDOW)
    item2d = item.astype(jnp.int32).reshape(
        BATCH // GATHER_WINDOW, GATHER_WINDOW)
    embs = [
        _sc_gather(user_factors, item_factors, user2d, item2d, c)
        for c in range(NUM_CHUNKS)
    ]
    outs = [
        _tc_mlp(ue, ie, W1, b1, W2, b2, W3, b3) for ue, ie in embs
    ]
    return jnp.concatenate(outs, axis=0)


# speedup vs baseline: 1.1179x; 1.0215x over previous
import functools

import jax
import jax.numpy as jnp
from jax import lax
from jax.experimental import pallas as pl
from jax.experimental.pallas import tpu as pltpu
from jax.experimental.pallas import tpu_sc as plsc

BATCH = 16384
D = 128
GATHER_WINDOW = 128
NUM_CHUNKS = 2
WINDOWS_PER_STEP = 2
CHUNK = BATCH // NUM_CHUNKS
MLP_BLOCK = 2048


NUM_CORES = 2
NUM_SUBCORES = 16
NUM_WORKERS = NUM_CORES * NUM_SUBCORES
ROWS_PER_WORKER = CHUNK // NUM_WORKERS
WINS_PER_WORKER = ROWS_PER_WORKER // GATHER_WINDOW


def _gather_body(chunk, uf_hbm, if_hbm, ui_hbm, ii_hbm, ue_hbm, ie_hbm,
                 idx_u, idx_i, ru, ri, gsem, wsem):
    w = GATHER_WINDOW
    wid = lax.axis_index("s") * NUM_CORES + lax.axis_index("c")
    base_win = chunk * (CHUNK // w) + wid * WINS_PER_WORKER
    pltpu.sync_copy(ui_hbm.at[pl.ds(base_win, WINS_PER_WORKER)], idx_u)
    pltpu.sync_copy(ii_hbm.at[pl.ds(base_win, WINS_PER_WORKER)], idx_i)
    gu, gi = [], []
    for j in range(WINS_PER_WORKER):
        gu.append(pltpu.make_async_copy(
            uf_hbm.at[idx_u.at[j]], ru.at[pl.ds(j * w, w)], gsem.at[2 * j]))
        gi.append(pltpu.make_async_copy(
            if_hbm.at[idx_i.at[j]], ri.at[pl.ds(j * w, w)],
            gsem.at[2 * j + 1]))
    for c in gu + gi:
        c.start()
    base_row = wid * ROWS_PER_WORKER
    for c in gu:
        c.wait()
    wu = pltpu.make_async_copy(
        ru, ue_hbm.at[pl.ds(base_row, ROWS_PER_WORKER)], wsem.at[0])
    wu.start()
    for c in gi:
        c.wait()
    wi = pltpu.make_async_copy(
        ri, ie_hbm.at[pl.ds(base_row, ROWS_PER_WORKER)], wsem.at[1])
    wi.start()
    wu.wait()
    wi.wait()


def _sc_gather(user_factors, item_factors, user2d, item2d, chunk):
    mesh = plsc.VectorSubcoreMesh(core_axis_name="c", subcore_axis_name="s")
    f = pl.kernel(
        functools.partial(_gather_body, chunk),
        out_type=(
            jax.ShapeDtypeStruct((CHUNK, D), jnp.float32),
            jax.ShapeDtypeStruct((CHUNK, D), jnp.float32),
        ),
        mesh=mesh,
        scratch_types=[
            pltpu.VMEM((WINS_PER_WORKER, GATHER_WINDOW), jnp.int32),
            pltpu.VMEM((WINS_PER_WORKER, GATHER_WINDOW), jnp.int32),
            pltpu.VMEM((ROWS_PER_WORKER, D), jnp.float32),
            pltpu.VMEM((ROWS_PER_WORKER, D), jnp.float32),
            pltpu.SemaphoreType.DMA((2 * WINS_PER_WORKER,)),
            pltpu.SemaphoreType.DMA((2,)),
        ],
    )
    return f(user_factors, item_factors, user2d, item2d)


def _mlp_body(ue_ref, ie_ref, w1u_ref, w1i_ref, b1_ref, w2_ref, b2_ref,
              w3_ref, b3_ref, o_ref):
    dn = (((1,), (1,)), ((), ()))
    h = lax.dot_general(ue_ref[...], w1u_ref[...], dn,
                        preferred_element_type=jnp.float32)
    h = h + lax.dot_general(ie_ref[...], w1i_ref[...], dn,
                            preferred_element_type=jnp.float32)
    h = jnp.maximum(h + b1_ref[...][None, :], 0.0)
    h = lax.dot_general(h, w2_ref[...], dn, preferred_element_type=jnp.float32)
    h = jnp.maximum(h + b2_ref[...][None, :], 0.0)
    h = lax.dot_general(h, w3_ref[...], dn, preferred_element_type=jnp.float32)
    o_ref[...] = jax.nn.sigmoid(h + b3_ref[...][None, :])


def _tc_mlp(ue, ie, W1, b1, W2, b2, W3, b3):
    n = ue.shape[0]
    blk = min(MLP_BLOCK, n)
    full = lambda shape: pl.BlockSpec(shape, lambda i: tuple(0 for _ in shape))
    return pl.pallas_call(
        _mlp_body,
        grid=(n // blk,),
        in_specs=[
            pl.BlockSpec((blk, D), lambda i: (i, 0)),
            pl.BlockSpec((blk, D), lambda i: (i, 0)),
            pl.BlockSpec((D, D), lambda i: (0, 0)),
        pl.BlockSpec((D, D), lambda i: (0, 1)),
        full(b1.shape),
            full(W2.shape), full(b2.shape), full(W3.shape), full(b3.shape),
        ],
        out_specs=pl.BlockSpec((blk, 2), lambda i: (i, 0)),
        out_shape=jax.ShapeDtypeStruct((n, 2), jnp.float32),
    )(ue, ie, W1, W1, b1, W2, b2, W3, b3)


def kernel(user, item, user_factors, item_factors, W1, b1, W2, b2, W3, b3):
    user2d = user.astype(jnp.int32).reshape(
        BATCH // GATHER_WINDOW, GATHER_WINDOW)
    item2d = item.astype(jnp.int32).reshape(
        BATCH // GATHER_WINDOW, GATHER_WINDOW)
    embs = [
        _sc_gather(user_factors, item_factors, user2d, item2d, c)
        for c in range(NUM_CHUNKS)
    ]
    outs = [
        _tc_mlp(ue, ie, W1, b1, W2, b2, W3, b3) for ue, ie in embs
    ]
    return jnp.concatenate(outs, axis=0)


# speedup vs baseline: 1.2394x; 1.1087x over previous
import functools

import jax
import jax.numpy as jnp
from jax import lax
from jax.experimental import pallas as pl
from jax.experimental.pallas import tpu as pltpu
from jax.experimental.pallas import tpu_sc as plsc

BATCH = 16384
D = 128
GATHER_WINDOW = 128
NUM_CHUNKS = 2
WINDOWS_PER_STEP = 2
CHUNK = BATCH // NUM_CHUNKS
MLP_BLOCK = 2048


NUM_CORES = 2
NUM_SUBCORES = 16
NUM_WORKERS = NUM_CORES * NUM_SUBCORES
ROWS_PER_WORKER = CHUNK // NUM_WORKERS
WINS_PER_WORKER = ROWS_PER_WORKER // GATHER_WINDOW


def _gather_body(chunk, uf_hbm, if_hbm, ui_hbm, ii_hbm, ue_hbm, ie_hbm,
                 idx_u, idx_i, ru, ri, gsem, wsem):
    w = GATHER_WINDOW
    wid = lax.axis_index("s") * NUM_CORES + lax.axis_index("c")
    base_win = chunk * (CHUNK // w) + wid * WINS_PER_WORKER
    pltpu.sync_copy(ui_hbm.at[pl.ds(base_win, WINS_PER_WORKER)], idx_u)
    pltpu.sync_copy(ii_hbm.at[pl.ds(base_win, WINS_PER_WORKER)], idx_i)
    gu, gi = [], []
    for j in range(WINS_PER_WORKER):
        gu.append(pltpu.make_async_copy(
            uf_hbm.at[idx_u.at[j]], ru.at[pl.ds(j * w, w)], gsem.at[2 * j]))
        gi.append(pltpu.make_async_copy(
            if_hbm.at[idx_i.at[j]], ri.at[pl.ds(j * w, w)],
            gsem.at[2 * j + 1]))
    for c in gu + gi:
        c.start()
    base_row = wid * ROWS_PER_WORKER
    for c in gu:
        c.wait()
    wu = pltpu.make_async_copy(
        ru, ue_hbm.at[pl.ds(base_row, ROWS_PER_WORKER)], wsem.at[0])
    wu.start()
    for c in gi:
        c.wait()
    wi = pltpu.make_async_copy(
        ri, ie_hbm.at[pl.ds(base_row, ROWS_PER_WORKER)], wsem.at[1])
    wi.start()
    wu.wait()
    wi.wait()


def _sc_gather(user_factors, item_factors, user2d, item2d, chunk):
    mesh = plsc.VectorSubcoreMesh(core_axis_name="c", subcore_axis_name="s")
    f = pl.kernel(
        functools.partial(_gather_body, chunk),
        out_type=(
            jax.ShapeDtypeStruct((CHUNK, D), jnp.float32),
            jax.ShapeDtypeStruct((CHUNK, D), jnp.float32),
        ),
        mesh=mesh,
        scratch_types=[
            pltpu.VMEM((WINS_PER_WORKER, GATHER_WINDOW), jnp.int32),
            pltpu.VMEM((WINS_PER_WORKER, GATHER_WINDOW), jnp.int32),
            pltpu.VMEM((ROWS_PER_WORKER, D), jnp.float32),
            pltpu.VMEM((ROWS_PER_WORKER, D), jnp.float32),
            pltpu.SemaphoreType.DMA((2 * WINS_PER_WORKER,)),
            pltpu.SemaphoreType.DMA((2,)),
        ],
    )
    return f(user_factors, item_factors, user2d, item2d)


def _mlp_body(ue_ref, ie_ref, w1u_ref, w1i_ref, b1_ref, w2_ref, b2_ref,
              w3_ref, b3_ref, o_ref):
    dn = (((1,), (1,)), ((), ()))
    h = lax.dot_general(ue_ref[...], w1u_ref[...], dn,
                        preferred_element_type=jnp.float32)
    h = h + lax.dot_general(ie_ref[...], w1i_ref[...], dn,
                            preferred_element_type=jnp.float32)
    h = jnp.maximum(h + b1_ref[...][None, :], 0.0)
    h = lax.dot_general(h, w2_ref[...], dn, preferred_element_type=jnp.float32)
    h = jnp.maximum(h + b2_ref[...][None, :], 0.0)
    h = lax.dot_general(w3_ref[...], h, dn, preferred_element_type=jnp.float32)
    o_ref[...] = jax.nn.sigmoid(h + b3_ref[...][:, None])


def _tc_mlp(ue, ie, W1, b1, W2, b2, W3, b3):
    n = ue.shape[0]
    blk = min(MLP_BLOCK, n)
    full = lambda shape: pl.BlockSpec(shape, lambda i: tuple(0 for _ in shape))
    return pl.pallas_call(
        _mlp_body,
        grid=(n // blk,),
        in_specs=[
            pl.BlockSpec((blk, D), lambda i: (i, 0)),
            pl.BlockSpec((blk, D), lambda i: (i, 0)),
            pl.BlockSpec((D, D), lambda i: (0, 0)),
        pl.BlockSpec((D, D), lambda i: (0, 1)),
        full(b1.shape),
            full(W2.shape), full(b2.shape), full(W3.shape), full(b3.shape),
        ],
        out_specs=pl.BlockSpec((2, blk), lambda i: (0, i)),
        out_shape=jax.ShapeDtypeStruct((2, n), jnp.float32),
    )(ue, ie, W1, W1, b1, W2, b2, W3, b3)


def kernel(user, item, user_factors, item_factors, W1, b1, W2, b2, W3, b3):
    user2d = user.astype(jnp.int32).reshape(
        BATCH // GATHER_WINDOW, GATHER_WINDOW)
    item2d = item.astype(jnp.int32).reshape(
        BATCH // GATHER_WINDOW, GATHER_WINDOW)
    embs = [
        _sc_gather(user_factors, item_factors, user2d, item2d, c)
        for c in range(NUM_CHUNKS)
    ]
    outs = [
        _tc_mlp(ue, ie, W1, b1, W2, b2, W3, b3) for ue, ie in embs
    ]
    return jnp.concatenate(outs, axis=1).T


# speedup vs baseline: 1.2459x; 1.0052x over previous
import functools

import jax
import jax.numpy as jnp
from jax import lax
from jax.experimental import pallas as pl
from jax.experimental.pallas import tpu as pltpu
from jax.experimental.pallas import tpu_sc as plsc

BATCH = 16384
D = 128
GATHER_WINDOW = 128
NUM_CHUNKS = 2
WINDOWS_PER_STEP = 2
CHUNK = BATCH // NUM_CHUNKS
MLP_BLOCK = 2048


NUM_CORES = 2
NUM_SUBCORES = 16
NUM_WORKERS = NUM_CORES * NUM_SUBCORES
ROWS_PER_WORKER = CHUNK // NUM_WORKERS
WINS_PER_WORKER = ROWS_PER_WORKER // GATHER_WINDOW


def _gather_body(uf_hbm, if_hbm, ui_hbm, ii_hbm, ue_hbm, ie_hbm,
                 idx_u, idx_i, ru, ri, gsem, wsem):
    w = GATHER_WINDOW
    wid = lax.axis_index("s") * NUM_CORES + lax.axis_index("c")
    base_win = wid * WINS_PER_WORKER
    pltpu.sync_copy(ui_hbm.at[pl.ds(base_win, WINS_PER_WORKER)], idx_u)
    pltpu.sync_copy(ii_hbm.at[pl.ds(base_win, WINS_PER_WORKER)], idx_i)
    gu, gi = [], []
    for j in range(WINS_PER_WORKER):
        gu.append(pltpu.make_async_copy(
            uf_hbm.at[idx_u.at[j]], ru.at[pl.ds(j * w, w)], gsem.at[2 * j]))
        gi.append(pltpu.make_async_copy(
            if_hbm.at[idx_i.at[j]], ri.at[pl.ds(j * w, w)],
            gsem.at[2 * j + 1]))
    for c in gu + gi:
        c.start()
    base_row = wid * ROWS_PER_WORKER
    for c in gu:
        c.wait()
    wu = pltpu.make_async_copy(
        ru, ue_hbm.at[pl.ds(base_row, ROWS_PER_WORKER)], wsem.at[0])
    wu.start()
    for c in gi:
        c.wait()
    wi = pltpu.make_async_copy(
        ri, ie_hbm.at[pl.ds(base_row, ROWS_PER_WORKER)], wsem.at[1])
    wi.start()
    wu.wait()
    wi.wait()


def _sc_gather(user_factors, item_factors, user2d, item2d):
    mesh = plsc.VectorSubcoreMesh(core_axis_name="c", subcore_axis_name="s")
    f = pl.kernel(
        _gather_body,
        out_type=(
            jax.ShapeDtypeStruct((CHUNK, D), jnp.float32),
            jax.ShapeDtypeStruct((CHUNK, D), jnp.float32),
        ),
        mesh=mesh,
        scratch_types=[
            pltpu.VMEM((WINS_PER_WORKER, GATHER_WINDOW), jnp.int32),
            pltpu.VMEM((WINS_PER_WORKER, GATHER_WINDOW), jnp.int32),
            pltpu.VMEM((ROWS_PER_WORKER, D), jnp.float32),
            pltpu.VMEM((ROWS_PER_WORKER, D), jnp.float32),
            pltpu.SemaphoreType.DMA((2 * WINS_PER_WORKER,)),
            pltpu.SemaphoreType.DMA((2,)),
        ],
    )
    return f(user_factors, item_factors, user2d, item2d)


def _mlp_body(ue_ref, ie_ref, w1u_ref, w1i_ref, b1_ref, w2_ref, b2_ref,
              w3_ref, b3_ref, o_ref):
    dn = (((1,), (1,)), ((), ()))
    h = lax.dot_general(ue_ref[...], w1u_ref[...], dn,
                        preferred_element_type=jnp.float32)
    h = h + lax.dot_general(ie_ref[...], w1i_ref[...], dn,
                            preferred_element_type=jnp.float32)
    h = jnp.maximum(h + b1_ref[...][None, :], 0.0)
    h = lax.dot_general(h, w2_ref[...], dn, preferred_element_type=jnp.float32)
    h = jnp.maximum(h + b2_ref[...][None, :], 0.0)
    h = lax.dot_general(w3_ref[...], h, dn, preferred_element_type=jnp.float32)
    o_ref[...] = jax.nn.sigmoid(h + b3_ref[...][:, None])


def _tc_mlp(ue, ie, W1, b1, W2, b2, W3, b3):
    n = ue.shape[0]
    blk = min(MLP_BLOCK, n)
    full = lambda shape: pl.BlockSpec(shape, lambda i: tuple(0 for _ in shape))
    return pl.pallas_call(
        _mlp_body,
        grid=(n // blk,),
        in_specs=[
            pl.BlockSpec((blk, D), lambda i: (i, 0)),
            pl.BlockSpec((blk, D), lambda i: (i, 0)),
            pl.BlockSpec((D, D), lambda i: (0, 0)),
        pl.BlockSpec((D, D), lambda i: (0, 1)),
        full(b1.shape),
            full(W2.shape), full(b2.shape), full(W3.shape), full(b3.shape),
        ],
        out_specs=pl.BlockSpec((2, blk), lambda i: (0, i)),
        out_shape=jax.ShapeDtypeStruct((2, n), jnp.float32),
    )(ue, ie, W1, W1, b1, W2, b2, W3, b3)


def kernel(user, item, user_factors, item_factors, W1, b1, W2, b2, W3, b3):
    user2d = user.astype(jnp.int32).reshape(
        NUM_CHUNKS, CHUNK // GATHER_WINDOW, GATHER_WINDOW)
    item2d = item.astype(jnp.int32).reshape(
        NUM_CHUNKS, CHUNK // GATHER_WINDOW, GATHER_WINDOW)
    embs = [
        _sc_gather(user_factors, item_factors, user2d[c], item2d[c])
        for c in range(NUM_CHUNKS)
    ]
    outs = [
        _tc_mlp(ue, ie, W1, b1, W2, b2, W3, b3) for ue, ie in embs
    ]
    return jnp.concatenate(outs, axis=1).T


# speedup vs baseline: 1.2581x; 1.0098x over previous
import functools

import jax
import jax.numpy as jnp
from jax import lax
from jax.experimental import pallas as pl
from jax.experimental.pallas import tpu as pltpu
from jax.experimental.pallas import tpu_sc as plsc

BATCH = 16384
D = 128
GATHER_WINDOW = 128
NUM_CHUNKS = 2
WINDOWS_PER_STEP = 2
CHUNK = BATCH // NUM_CHUNKS
MLP_BLOCK = 4096


NUM_CORES = 2
NUM_SUBCORES = 16
NUM_WORKERS = NUM_CORES * NUM_SUBCORES
ROWS_PER_WORKER = CHUNK // NUM_WORKERS
WINS_PER_WORKER = ROWS_PER_WORKER // GATHER_WINDOW


def _gather_body(uf_hbm, if_hbm, ui_hbm, ii_hbm, ue_hbm, ie_hbm,
                 idx_u, idx_i, ru, ri, gsem, wsem):
    w = GATHER_WINDOW
    wid = lax.axis_index("s") * NUM_CORES + lax.axis_index("c")
    base_win = wid * WINS_PER_WORKER
    pltpu.sync_copy(ui_hbm.at[pl.ds(base_win, WINS_PER_WORKER)], idx_u)
    pltpu.sync_copy(ii_hbm.at[pl.ds(base_win, WINS_PER_WORKER)], idx_i)
    gu, gi = [], []
    for j in range(WINS_PER_WORKER):
        gu.append(pltpu.make_async_copy(
            uf_hbm.at[idx_u.at[j]], ru.at[pl.ds(j * w, w)], gsem.at[2 * j]))
        gi.append(pltpu.make_async_copy(
            if_hbm.at[idx_i.at[j]], ri.at[pl.ds(j * w, w)],
            gsem.at[2 * j + 1]))
    for c in gu + gi:
        c.start()
    base_row = wid * ROWS_PER_WORKER
    for c in gu:
        c.wait()
    wu = pltpu.make_async_copy(
        ru, ue_hbm.at[pl.ds(base_row, ROWS_PER_WORKER)], wsem.at[0])
    wu.start()
    for c in gi:
        c.wait()
    wi = pltpu.make_async_copy(
        ri, ie_hbm.at[pl.ds(base_row, ROWS_PER_WORKER)], wsem.at[1])
    wi.start()
    wu.wait()
    wi.wait()


def _sc_gather(user_factors, item_factors, user2d, item2d):
    mesh = plsc.VectorSubcoreMesh(core_axis_name="c", subcore_axis_name="s")
    f = pl.kernel(
        _gather_body,
        out_type=(
            jax.ShapeDtypeStruct((CHUNK, D), jnp.float32),
            jax.ShapeDtypeStruct((CHUNK, D), jnp.float32),
        ),
        mesh=mesh,
        scratch_types=[
            pltpu.VMEM((WINS_PER_WORKER, GATHER_WINDOW), jnp.int32),
            pltpu.VMEM((WINS_PER_WORKER, GATHER_WINDOW), jnp.int32),
            pltpu.VMEM((ROWS_PER_WORKER, D), jnp.float32),
            pltpu.VMEM((ROWS_PER_WORKER, D), jnp.float32),
            pltpu.SemaphoreType.DMA((2 * WINS_PER_WORKER,)),
            pltpu.SemaphoreType.DMA((2,)),
        ],
    )
    return f(user_factors, item_factors, user2d, item2d)


def _mlp_body(ue_ref, ie_ref, w1u_ref, w1i_ref, b1_ref, w2_ref, b2_ref,
              w3_ref, b3_ref, o_ref):
    dn = (((1,), (1,)), ((), ()))
    h = lax.dot_general(ue_ref[...], w1u_ref[...], dn,
                        preferred_element_type=jnp.float32)
    h = h + lax.dot_general(ie_ref[...], w1i_ref[...], dn,
                            preferred_element_type=jnp.float32)
    h = jnp.maximum(h + b1_ref[...][None, :], 0.0)
    h = lax.dot_general(h, w2_ref[...], dn, preferred_element_type=jnp.float32)
    h = jnp.maximum(h + b2_ref[...][None, :], 0.0)
    h = lax.dot_general(w3_ref[...], h, dn, preferred_element_type=jnp.float32)
    o_ref[...] = jax.nn.sigmoid(h + b3_ref[...][:, None])


def _tc_mlp(ue, ie, W1, b1, W2, b2, W3, b3):
    n = ue.shape[0]
    blk = min(MLP_BLOCK, n)
    full = lambda shape: pl.BlockSpec(shape, lambda i: tuple(0 for _ in shape))
    return pl.pallas_call(
        _mlp_body,
        grid=(n // blk,),
        in_specs=[
            pl.BlockSpec((blk, D), lambda i: (i, 0)),
            pl.BlockSpec((blk, D), lambda i: (i, 0)),
            pl.BlockSpec((D, D), lambda i: (0, 0)),
        pl.BlockSpec((D, D), lambda i: (0, 1)),
        full(b1.shape),
            full(W2.shape), full(b2.shape), full(W3.shape), full(b3.shape),
        ],
        out_specs=pl.BlockSpec((2, blk), lambda i: (0, i)),
        out_shape=jax.ShapeDtypeStruct((2, n), jnp.float32),
    )(ue, ie, W1, W1, b1, W2, b2, W3, b3)


def kernel(user, item, user_factors, item_factors, W1, b1, W2, b2, W3, b3):
    user2d = user.astype(jnp.int32).reshape(
        NUM_CHUNKS, CHUNK // GATHER_WINDOW, GATHER_WINDOW)
    item2d = item.astype(jnp.int32).reshape(
        NUM_CHUNKS, CHUNK // GATHER_WINDOW, GATHER_WINDOW)
    embs = [
        _sc_gather(user_factors, item_factors, user2d[c], item2d[c])
        for c in range(NUM_CHUNKS)
    ]
    outs = [
        _tc_mlp(ue, ie, W1, b1, W2, b2, W3, b3) for ue, ie in embs
    ]
    return jnp.concatenate(outs, axis=1).T
